# baseline (speedup 1.0000x reference)
import functools

import jax
import jax.numpy as jnp
from jax import lax
from jax.experimental import pallas as pl
from jax.experimental.pallas import tpu as pltpu
from jax.experimental.pallas import tpu_sc as plsc

D_MODEL = 128
LANES = 16
NUM_WORKERS = 32
N_STRIPES = 8
CHUNK = 80
NBUF = 4
TC_BLK = 1024


@functools.partial(jax.jit, static_argnames=("n_rows",))
def _sc_gather_stripe(y2, pe, n_rows):
    n_per_w = n_rows // NUM_WORKERS
    n_chunks = n_per_w // CHUNK
    pd = NBUF - 1

    mesh = plsc.VectorSubcoreMesh(core_axis_name="c", subcore_axis_name="s")

    @functools.partial(
        pl.kernel,
        out_type=jax.ShapeDtypeStruct((n_rows, D_MODEL), jnp.float32),
        mesh=mesh,
        scratch_types=(
            [pltpu.VMEM((n_chunks, CHUNK), jnp.int32)]
            + [pltpu.VMEM((CHUNK, D_MODEL), jnp.float32)] * NBUF
            + [pltpu.SemaphoreType.DMA] * (2 * NBUF)
        ),
    )
    def run(y_hbm, pe_hbm, out_hbm, idx_v, *rest):
        pbufs = rest[0:NBUF]
        in_sems = rest[NBUF:2 * NBUF]
        out_sems = rest[2 * NBUF:3 * NBUF]

        wid = lax.axis_index("s") * 2 + lax.axis_index("c")
        row0 = wid * n_per_w

        pltpu.sync_copy(y_hbm.at[wid], idx_v)

        def start_in(j, b):
            pltpu.make_async_copy(
                pe_hbm.at[idx_v.at[j]], pbufs[b], in_sems[b]
            ).start()

        def wait_in(b):
            pltpu.make_async_copy(
                pe_hbm.at[pl.ds(0, CHUNK)], pbufs[b], in_sems[b]
            ).wait()

        def start_out(j, b):
            pltpu.make_async_copy(
                pbufs[b], out_hbm.at[pl.ds(row0 + j * CHUNK, CHUNK)],
                out_sems[b]
            ).start()

        def wait_out(b):
            pltpu.make_async_copy(
                pbufs[b], out_hbm.at[pl.ds(0, CHUNK)], out_sems[b]
            ).wait()

        def step(j, b, wait_prev):
            bp = (b + pd) % NBUF
            wait_in(b)
            start_out(j, b)
            if wait_prev is True:
                wait_out(bp)
            elif wait_prev is not False:
                @pl.when(wait_prev)
                def _():
                    wait_out(bp)
            start_in(j + pd, bp)

        for j in range(pd):
            start_in(j, j)

        n_main = (n_chunks - pd) // NBUF * NBUF

        def outer(g, carry):
            for i in range(NBUF):
                j = g * NBUF + i
                step(j, i, wait_prev=(j >= 1) if i == 0 else True)
            return carry

        lax.fori_loop(0, n_main // NBUF, outer, 0)

        for j in range(n_main, n_chunks):
            b = j % NBUF
            bp = (b + pd) % NBUF
            wait_in(b)
            start_out(j, b)
            if j + pd < n_chunks:
                wait_out(bp)
                start_in(j + pd, bp)

        for j in range(max(0, n_chunks - NBUF), n_chunks):
            wait_out(j % NBUF)

    return run(y2, pe)


def _tc_add_stripe(o, g, stripe_base):
    n_blk = g.shape[0] // TC_BLK
    base_blk = stripe_base // TC_BLK

    def body(o_ref, g_ref, out_ref):
        out_ref[...] = o_ref[...] + g_ref[...]

    return pl.pallas_call(
        body,
        grid=(n_blk,),
        in_specs=[
            pl.BlockSpec((TC_BLK, D_MODEL), lambda i: (base_blk + i, 0)),
            pl.BlockSpec((TC_BLK, D_MODEL), lambda i: (i, 0)),
        ],
        out_specs=pl.BlockSpec((TC_BLK, D_MODEL), lambda i: (base_blk + i, 0)),
        out_shape=jax.ShapeDtypeStruct(o.shape, o.dtype),
        input_output_aliases={0: 0},
    )(o, g)


def kernel(x, y, pe):
    seq, batch, d = x.shape
    n_rows = seq * batch
    stripe = n_rows // N_STRIPES
    x2 = x.reshape(n_rows, d)
    y_flat = y.reshape(-1).astype(jnp.int32)
    y_stripes = y_flat.reshape(
        N_STRIPES, NUM_WORKERS, stripe // (NUM_WORKERS * CHUNK), CHUNK)

    gathered = [
        _sc_gather_stripe(y_stripes[k], pe, stripe) for k in range(N_STRIPES)
    ]
    o = x2
    for k in range(N_STRIPES):
        o = _tc_add_stripe(o, gathered[k], k * stripe)
    return o.reshape(x.shape)


# speedup vs baseline: 2.4463x; 2.4463x over previous
import functools

import jax
import jax.numpy as jnp
from jax import lax
from jax.experimental import pallas as pl
from jax.experimental.pallas import tpu as pltpu
from jax.experimental.pallas import tpu_sc as plsc

D_MODEL = 128
LANES = 16
NUM_WORKERS = 32
CHUNK = 64
NBUF = 4


@functools.partial(jax.jit, static_argnames=("n_rows",))
def _sc_gather_add(x2, y2, pe, n_rows):
    n_per_w = n_rows // NUM_WORKERS
    n_chunks = n_per_w // CHUNK
    pd = NBUF - 1

    mesh = plsc.VectorSubcoreMesh(core_axis_name="c", subcore_axis_name="s")

    @functools.partial(
        pl.kernel,
        out_type=jax.ShapeDtypeStruct((n_rows, D_MODEL), jnp.float32),
        mesh=mesh,
        scratch_types=(
            [pltpu.VMEM((n_chunks, CHUNK), jnp.int32)]
            + [pltpu.VMEM((CHUNK, D_MODEL), jnp.float32)] * (2 * NBUF)
            + [pltpu.VMEM_SHARED((16, NBUF, CHUNK, D_MODEL), jnp.float32)]
            + [pltpu.SemaphoreType.DMA] * (2 * NBUF)
        ),
    )
    def run(x_hbm, y_hbm, pe_hbm, out_hbm, idx_v, *rest):
        xbufs = rest[0:NBUF]
        pbufs = rest[NBUF:2 * NBUF]
        shared = rest[2 * NBUF]
        in_sems = rest[2 * NBUF + 1:3 * NBUF + 1]
        out_sems = rest[3 * NBUF + 1:4 * NBUF + 1]

        sid = lax.axis_index("s")
        wid = sid * 2 + lax.axis_index("c")
        row0 = wid * n_per_w

        pltpu.sync_copy(y_hbm.at[wid], idx_v)

        def start_in(j, b):
            pltpu.make_async_copy(
                x_hbm.at[pl.ds(row0 + j * CHUNK, CHUNK)], xbufs[b], in_sems[b]
            ).start()
            pltpu.make_async_copy(
                pe_hbm.at[idx_v.at[j]], pbufs[b], in_sems[b]
            ).start()

        def wait_in(b):
            pltpu.make_async_copy(
                x_hbm.at[pl.ds(0, CHUNK)], xbufs[b], in_sems[b]
            ).wait()
            pltpu.make_async_copy(
                x_hbm.at[pl.ds(0, CHUNK)], pbufs[b], in_sems[b]
            ).wait()

        def start_out(j, b):
            pltpu.make_async_copy(
                shared.at[sid, b], out_hbm.at[pl.ds(row0 + j * CHUNK, CHUNK)],
                out_sems[b]
            ).start()

        def wait_out(b):
            pltpu.make_async_copy(
                shared.at[sid, b], out_hbm.at[pl.ds(0, CHUNK)], out_sems[b]
            ).wait()

        def compute(b):
            xr, pr = xbufs[b], pbufs[b]

            @plsc.parallel_loop(0, CHUNK, unroll=4)
            def row_body(r):
                for c in range(D_MODEL // LANES):
                    sl = pl.ds(c * LANES, LANES)
                    xr[r, sl] = xr[r, sl] + pr[r, sl]

        def step(j, b, wait_prev):
            bp = (b + pd) % NBUF
            wait_in(b)
            compute(b)
            if wait_prev is True:
                wait_out(b)
            elif wait_prev is not False:
                @pl.when(wait_prev)
                def _():
                    wait_out(b)
            pltpu.sync_copy(xbufs[b], shared.at[sid, b])
            start_out(j, b)
            start_in(j + pd, bp)

        for j in range(pd):
            start_in(j, j)

        n_main = (n_chunks - pd) // NBUF * NBUF

        def outer(g, carry):
            for i in range(NBUF):
                j = g * NBUF + i
                step(j, i, wait_prev=(g >= 1))
            return carry

        lax.fori_loop(0, n_main // NBUF, outer, 0)

        for j in range(n_main, n_chunks):
            b = j % NBUF
            bp = (b + pd) % NBUF
            wait_in(b)
            compute(b)
            wait_out(b)
            pltpu.sync_copy(xbufs[b], shared.at[sid, b])
            start_out(j, b)
            if j + pd < n_chunks:
                start_in(j + pd, bp)

        for j in range(max(0, n_chunks - NBUF), n_chunks):
            wait_out(j % NBUF)

    return run(x2, y2, pe)


def kernel(x, y, pe):
    seq, batch, d = x.shape
    n_rows = seq * batch
    x2 = x.reshape(n_rows, d)
    y2 = y.reshape(-1).astype(jnp.int32).reshape(
        NUM_WORKERS, n_rows // (NUM_WORKERS * CHUNK), CHUNK)
    out = _sc_gather_add(x2, y2, pe, n_rows)
    return out.reshape(x.shape)
